# baseline (speedup 1.0000x reference)
import functools

import jax
import jax.numpy as jnp
from jax import lax
from jax.experimental import pallas as pl
from jax.experimental.pallas import tpu as pltpu
from jax.experimental.pallas import tpu_sc as plsc

B = 16384
D = 64
NC = 2
NS = 16
L = 16
NW = NC * NS
BPW = B // NW
NCHUNK = 4
CHUNK = BPW // NCHUNK
NG = BPW // L


def _mf_body(uid_hbm, iid_hbm, utab_hbm, itab_hbm, ub_hbm, ib_hbm, out_hbm,
             idx_u, idx_i, rows_u, rows_i, bias_u, bias_i, out_v, sem):
    wid = lax.axis_index("s") * NC + lax.axis_index("c")
    base = wid * BPW

    pltpu.sync_copy(uid_hbm.at[wid], idx_u)
    pltpu.sync_copy(iid_hbm.at[wid], idx_i)

    copies = []
    for j in range(NCHUNK):
        copies.append(pltpu.async_copy(
            utab_hbm.at[idx_u.at[j]],
            rows_u.at[pl.ds(j * CHUNK, CHUNK)], sem))
        copies.append(pltpu.async_copy(
            itab_hbm.at[idx_i.at[j]],
            rows_i.at[pl.ds(j * CHUNK, CHUNK)], sem))
        copies.append(pltpu.async_copy(
            ub_hbm.at[idx_u.at[j]], bias_u.at[pl.ds(j * CHUNK, CHUNK)], sem))
        copies.append(pltpu.async_copy(
            ib_hbm.at[idx_i.at[j]], bias_i.at[pl.ds(j * CHUNK, CHUNK)], sem))
    for c in copies:
        c.wait()

    lanes = lax.iota(jnp.int32, L)

    def group(g, carry):
        row = g * L + lanes
        acc = bias_u[pl.ds(g * L, L)] + bias_i[pl.ds(g * L, L)]
        for d in range(D):
            col = jnp.full((L,), d, jnp.int32)
            u = plsc.load_gather(rows_u, [row, col])
            v = plsc.load_gather(rows_i, [row, col])
            acc = acc + u * v
        out_v[pl.ds(g * L, L)] = acc
        return carry

    lax.fori_loop(0, NG, group, 0)

    pltpu.sync_copy(out_v, out_hbm.at[pl.ds(base, BPW)])


@functools.partial(jax.jit, static_argnames=())
def _mf(user_ids, item_ids, utab, itab, ub_flat, ib_flat):
    mesh = plsc.VectorSubcoreMesh(core_axis_name="c", subcore_axis_name="s")
    kern = functools.partial(
        pl.kernel,
        out_type=jax.ShapeDtypeStruct((B,), jnp.float32),
        mesh=mesh,
        scratch_types=[
            pltpu.VMEM((NCHUNK, CHUNK), jnp.int32),
            pltpu.VMEM((NCHUNK, CHUNK), jnp.int32),
            pltpu.VMEM((BPW, D), jnp.float32),
            pltpu.VMEM((BPW, D), jnp.float32),
            pltpu.VMEM((BPW,), jnp.float32),
            pltpu.VMEM((BPW,), jnp.float32),
            pltpu.VMEM((BPW,), jnp.float32),
            pltpu.SemaphoreType.DMA,
        ],
        compiler_params=pltpu.CompilerParams(
            needs_layout_passes=False, use_tc_tiling_on_sc=False),
    )(_mf_body)
    return kern(user_ids, item_ids, utab, itab, ub_flat, ib_flat)


def kernel(user_ids, item_ids, user_emb_table, item_emb_table,
           user_bias_table, item_bias_table):
    uid = user_ids.astype(jnp.int32).reshape(NW, NCHUNK, CHUNK)
    iid = item_ids.astype(jnp.int32).reshape(NW, NCHUNK, CHUNK)
    ub = user_bias_table.reshape(-1)
    ib = item_bias_table.reshape(-1)
    return _mf(uid, iid, user_emb_table, item_emb_table, ub, ib)
